# Initial kernel scaffold; baseline (speedup 1.0000x reference)
import jax
import jax.numpy as jnp
from jax.experimental import pallas as pl


def kernel(x, edge_index, batch, W1, att_src1, att_dst1, b1, W2, att_src2, att_dst2, b2, Wl, bl):
    raise NotImplementedError("write your pallas kernel here")


# speedup vs baseline: 1.0147x; 1.0147x over previous
import functools
import jax
import jax.numpy as jnp
from jax import lax
from jax.experimental import pallas as pl
from jax.experimental.pallas import tpu as pltpu

N_NODES = 10000
IN_DIM = 128
HID = 64
HEAD = 8
FEAT = HID * HEAD
CLASSES = 10
N_GRAPHS = 64

NODE_TILE = 1000
EDGE_BLK = 512


def _dense_body(x_ref, w_ref, asrc_ref, adst_ref, h_ref, ha_ref, hb_ref):
    h = jnp.dot(x_ref[...], w_ref[...], preferred_element_type=jnp.float32)
    h_ref[...] = h
    ha_ref[...] = jnp.dot(h, asrc_ref[...], preferred_element_type=jnp.float32)
    hb_ref[...] = jnp.dot(h, adst_ref[...], preferred_element_type=jnp.float32)


def _dense(x, w, a_src, a_dst):
    n, d_in = x.shape
    grid = n // NODE_TILE
    return pl.pallas_call(
        _dense_body,
        grid=(grid,),
        in_specs=[
            pl.BlockSpec((NODE_TILE, d_in), lambda i: (i, 0)),
            pl.BlockSpec((d_in, FEAT), lambda i: (0, 0)),
            pl.BlockSpec((FEAT, HEAD), lambda i: (0, 0)),
            pl.BlockSpec((FEAT, HEAD), lambda i: (0, 0)),
        ],
        out_specs=[
            pl.BlockSpec((NODE_TILE, FEAT), lambda i: (i, 0)),
            pl.BlockSpec((NODE_TILE, HEAD), lambda i: (i, 0)),
            pl.BlockSpec((NODE_TILE, HEAD), lambda i: (i, 0)),
        ],
        out_shape=[
            jax.ShapeDtypeStruct((n, FEAT), jnp.float32),
            jax.ShapeDtypeStruct((n, HEAD), jnp.float32),
            jax.ShapeDtypeStruct((n, HEAD), jnp.float32),
        ],
    )(x, w, a_src, a_dst)


def _edge_body(src_ref, dst_ref, h_ref, asrc_ref, adst_ref, t8_ref,
               numer_ref, denom_ref):
    step = pl.program_id(0)

    @pl.when(step == 0)
    def _init():
        alpha = asrc_ref[...] + adst_ref[...]
        alpha = jnp.where(alpha > 0, alpha, 0.2 * alpha)
        ex = jnp.exp(alpha)
        denom_ref[...] = ex
        e512 = jnp.dot(ex, t8_ref[...], preferred_element_type=jnp.float32)
        numer_ref[...] = h_ref[...] * e512

    def body(i, _):
        s = src_ref[i]
        d = dst_ref[i]
        arow = asrc_ref[pl.ds(s, 1), :]
        brow = adst_ref[pl.ds(d, 1), :]
        alpha = arow + brow
        alpha = jnp.where(alpha > 0, alpha, 0.2 * alpha)
        ex = jnp.exp(alpha)
        denom_ref[pl.ds(d, 1), :] = denom_ref[pl.ds(d, 1), :] + ex
        e512 = jnp.dot(ex, t8_ref[...], preferred_element_type=jnp.float32)
        hrow = h_ref[pl.ds(s, 1), :]
        numer_ref[pl.ds(d, 1), :] = numer_ref[pl.ds(d, 1), :] + hrow * e512
        return 0

    lax.fori_loop(0, EDGE_BLK, body, 0)


def _edge_pass(src, dst, h, asrc, adst, t8):
    n = h.shape[0]
    grid = src.shape[0] // EDGE_BLK
    return pl.pallas_call(
        _edge_body,
        grid=(grid,),
        in_specs=[
            pl.BlockSpec((EDGE_BLK,), lambda i: (i,),
                         memory_space=pltpu.MemorySpace.SMEM),
            pl.BlockSpec((EDGE_BLK,), lambda i: (i,),
                         memory_space=pltpu.MemorySpace.SMEM),
            pl.BlockSpec((n, FEAT), lambda i: (0, 0)),
            pl.BlockSpec((n, HEAD), lambda i: (0, 0)),
            pl.BlockSpec((n, HEAD), lambda i: (0, 0)),
            pl.BlockSpec((HEAD, FEAT), lambda i: (0, 0)),
        ],
        out_specs=[
            pl.BlockSpec((n, FEAT), lambda i: (0, 0)),
            pl.BlockSpec((n, HEAD), lambda i: (0, 0)),
        ],
        out_shape=[
            jax.ShapeDtypeStruct((n, FEAT), jnp.float32),
            jax.ShapeDtypeStruct((n, HEAD), jnp.float32),
        ],
    )(src, dst, h, asrc, adst, t8)


def _norm_body(numer_ref, denom_ref, t8_ref, b_ref, out_ref):
    d512 = jnp.dot(denom_ref[...], t8_ref[...],
                   preferred_element_type=jnp.float32)
    v = numer_ref[...] / (d512 + 1e-16) + b_ref[...]
    out_ref[...] = jnp.where(v > 0, v, jnp.exp(jnp.minimum(v, 0.0)) - 1.0)


def _normalize_elu(numer, denom, t8, b):
    n = numer.shape[0]
    grid = n // NODE_TILE
    return pl.pallas_call(
        _norm_body,
        grid=(grid,),
        in_specs=[
            pl.BlockSpec((NODE_TILE, FEAT), lambda i: (i, 0)),
            pl.BlockSpec((NODE_TILE, HEAD), lambda i: (i, 0)),
            pl.BlockSpec((HEAD, FEAT), lambda i: (0, 0)),
            pl.BlockSpec((1, FEAT), lambda i: (0, 0)),
        ],
        out_specs=pl.BlockSpec((NODE_TILE, FEAT), lambda i: (i, 0)),
        out_shape=jax.ShapeDtypeStruct((n, FEAT), jnp.float32),
    )(numer, denom, t8, b)


def _pool_body(h_ref, batch_ref, wl_ref, bl_ref, out_ref, sums_ref, cnt_ref):
    step = pl.program_id(0)
    nsteps = pl.num_programs(0)

    @pl.when(step == 0)
    def _init():
        sums_ref[...] = jnp.zeros_like(sums_ref)
        cnt_ref[...] = jnp.zeros_like(cnt_ref)

    gid = lax.broadcasted_iota(jnp.int32, (NODE_TILE, N_GRAPHS), 1)
    oh = (batch_ref[...] == gid).astype(jnp.float32)
    sums_ref[...] += lax.dot_general(
        oh, h_ref[...], (((0,), (0,)), ((), ())),
        preferred_element_type=jnp.float32)
    cnt_ref[...] += lax.dot_general(
        oh, jnp.ones((NODE_TILE, 1), jnp.float32), (((0,), (0,)), ((), ())),
        preferred_element_type=jnp.float32)

    @pl.when(step == nsteps - 1)
    def _final():
        pooled = sums_ref[...] / jnp.maximum(cnt_ref[...], 1.0)
        logits = jnp.dot(pooled, wl_ref[...],
                         preferred_element_type=jnp.float32) + bl_ref[...]
        m = jnp.max(logits, axis=1, keepdims=True)
        lse = m + jnp.log(jnp.sum(jnp.exp(logits - m), axis=1, keepdims=True))
        out_ref[...] = logits - lse


def _pool_classify(h, batch2d, wl, bl):
    n = h.shape[0]
    grid = n // NODE_TILE
    return pl.pallas_call(
        _pool_body,
        grid=(grid,),
        in_specs=[
            pl.BlockSpec((NODE_TILE, FEAT), lambda i: (i, 0)),
            pl.BlockSpec((NODE_TILE, 1), lambda i: (i, 0)),
            pl.BlockSpec((FEAT, CLASSES), lambda i: (0, 0)),
            pl.BlockSpec((1, CLASSES), lambda i: (0, 0)),
        ],
        out_specs=pl.BlockSpec((N_GRAPHS, CLASSES), lambda i: (0, 0)),
        out_shape=jax.ShapeDtypeStruct((N_GRAPHS, CLASSES), jnp.float32),
        scratch_shapes=[
            pltpu.VMEM((N_GRAPHS, FEAT), jnp.float32),
            pltpu.VMEM((N_GRAPHS, 1), jnp.float32),
        ],
    )(h, batch2d, wl, bl)


def _att_mat(att):
    eye = jnp.eye(HEAD, dtype=jnp.float32)
    return (att[:, :, None] * eye[:, None, :]).reshape(FEAT, HEAD)


def kernel(x, edge_index, batch, W1, att_src1, att_dst1, b1,
           W2, att_src2, att_dst2, b2, Wl, bl):
    narange = jnp.arange(FEAT)
    perm = (narange % HEAD) * HID + narange // HEAD

    W1p = W1[:, perm]
    W2p = W2[perm][:, perm]
    Wlp = Wl[perm]
    b1p = b1[perm].reshape(1, FEAT)
    b2p = b2[perm].reshape(1, FEAT)
    A_src1 = _att_mat(att_src1)[perm]
    A_dst1 = _att_mat(att_dst1)[perm]
    A_src2 = _att_mat(att_src2)[perm]
    A_dst2 = _att_mat(att_dst2)[perm]
    t8 = (narange[None, :] % HEAD == jnp.arange(HEAD)[:, None]).astype(
        jnp.float32)

    src = edge_index[0]
    dst = edge_index[1]
    bl2 = bl.reshape(1, CLASSES)
    batch2d = batch.reshape(N_NODES, 1)

    h1, asrc1, adst1 = _dense(x, W1p, A_src1, A_dst1)
    numer1, denom1 = _edge_pass(src, dst, h1, asrc1, adst1, t8)
    z1 = _normalize_elu(numer1, denom1, t8, b1p)

    h2, asrc2, adst2 = _dense(z1, W2p, A_src2, A_dst2)
    numer2, denom2 = _edge_pass(src, dst, h2, asrc2, adst2, t8)
    z2 = _normalize_elu(numer2, denom2, t8, b2p)

    return _pool_classify(z2, batch2d, Wlp, bl2)
